# baseline (speedup 1.0000x reference)
import functools

import jax
import jax.numpy as jnp
from jax import lax
from jax.experimental import pallas as pl
from jax.experimental.pallas import tpu as pltpu
from jax.experimental.pallas import tpu_sc as plsc


def _detile_body(tt_ref, eye_ref, out_ref):
    tt = tt_ref[...]
    e, blk = tt.shape
    y = lax.dot_general(tt, eye_ref[...], (((0,), (0,)), ((), ())),
                        preferred_element_type=jnp.float32)
    y3 = y.reshape(blk // 8, 8, e)
    pieces = [y3[:, c, :] for c in range(8)]
    out_ref[...] = jnp.concatenate(pieces, axis=1)


def _detile(tableT):
    e, v = tableT.shape
    blk = 4096
    grid = (v + blk - 1) // blk
    return pl.pallas_call(
        _detile_body,
        grid=(grid,),
        in_specs=[pl.BlockSpec((e, blk), lambda i: (0, i)),
                  pl.BlockSpec((e, e), lambda i: (0, 0))],
        out_specs=pl.BlockSpec((blk * e // 128, 128), lambda i: (i, 0)),
        out_shape=jax.ShapeDtypeStruct((v * e // 128, 128), jnp.float32),
        compiler_params=pltpu.CompilerParams(fuse_transposed_lhs_in_matmul=True),
    )(tableT, jnp.eye(e, dtype=jnp.float32))


def _sc_detile(tableT):
    e, v = tableT.shape
    n_rows = v * e // 128
    info = plsc.get_sparse_core_info()
    nw = info.num_cores * info.num_subcores
    main_rows = n_rows // 16 * 16
    tail_rows = n_rows - main_rows
    rows_per = (main_rows // nw) // 16 * 16
    chunk_rows = 240
    mesh = plsc.VectorSubcoreMesh(core_axis_name="c", subcore_axis_name="s")

    @functools.partial(
        pl.kernel,
        mesh=mesh,
        out_type=jax.ShapeDtypeStruct((n_rows, 128), jnp.float32),
        scratch_types=[
            pltpu.VMEM((e, chunk_rows * 8), jnp.float32),
            pltpu.VMEM((chunk_rows, 128), jnp.float32),
            pltpu.VMEM((e, max(tail_rows, 1) * 8), jnp.float32),
        ],
        compiler_params=pltpu.CompilerParams(needs_layout_passes=False),
    )
    def k(tt_hbm, tail_hbm, out_hbm, tt_v, pk_v, tail_v):
        wid = lax.axis_index("s") * info.num_cores + lax.axis_index("c")
        base = wid * rows_per
        my_rows = jnp.where(wid == nw - 1, main_rows - base, rows_per)
        n_ch = (my_rows + chunk_rows - 1) // chunk_rows
        dims = lax.broadcasted_iota(jnp.int32, (16,), 0)

        def chunk_body(c, _):
            row0 = base + jnp.minimum(c * chunk_rows, my_rows - chunk_rows)
            pltpu.sync_copy(tt_hbm.at[:, pl.ds(row0 * 8, chunk_rows * 8)],
                            tt_v)

            def v_body(vl, _):
                row = plsc.load_gather(
                    tt_v, [dims, jnp.full((16,), vl, jnp.int32)])
                pk_v[vl // 8, pl.ds((vl % 8) * e, e)] = row
                return 0

            lax.fori_loop(0, chunk_rows * 8, v_body, 0, unroll=8)
            pltpu.sync_copy(pk_v, out_hbm.at[pl.ds(row0, chunk_rows)])
            return 0

        lax.fori_loop(0, n_ch, chunk_body, 0)

        if tail_rows:
            @pl.when(wid == 0)
            def _tail():
                pltpu.sync_copy(tail_hbm, tail_v)

                def t_body(vl, _):
                    row = plsc.load_gather(
                        tail_v, [dims, jnp.full((16,), vl, jnp.int32)])
                    pk_v[vl // 8, pl.ds((vl % 8) * e, e)] = row
                    return 0

                lax.fori_loop(0, tail_rows * 8, t_body, 0, unroll=8)
                pltpu.sync_copy(pk_v.at[pl.ds(0, tail_rows)],
                                out_hbm.at[pl.ds(main_rows, tail_rows)])

    tail = lax.slice(tableT, (0, main_rows * 8),
                     (e, main_rows * 8 + max(tail_rows, 1) * 8))
    return k(tableT, tail)


def _sc_gather(table, idx_flat):
    n_total = idx_flat.shape[0]
    v, e = table.shape
    info = plsc.get_sparse_core_info()
    num_cores, num_subcores = info.num_cores, info.num_subcores
    nw = num_cores * num_subcores
    per_w = n_total // nw
    chunk = 4096
    while per_w % chunk:
        chunk //= 2
    n_ch = per_w // chunk

    mesh = plsc.VectorSubcoreMesh(core_axis_name="c", subcore_axis_name="s")

    @functools.partial(
        pl.kernel,
        mesh=mesh,
        out_type=jax.ShapeDtypeStruct((n_total, e), jnp.float32),
        scratch_types=[
            pltpu.VMEM((chunk,), jnp.int32),
            pltpu.VMEM((chunk, e), jnp.float32),
            pltpu.SemaphoreType.DMA,
        ],
        compiler_params=pltpu.CompilerParams(use_tc_tiling_on_sc=False),
    )
    def k(table_hbm, idx_hbm, out_hbm, idx_v, rows_v, sem):
        wid = lax.axis_index("s") * num_cores + lax.axis_index("c")
        base = wid * per_w
        for c in range(n_ch):
            off = base + c * chunk
            pltpu.sync_copy(idx_hbm.at[pl.ds(off, chunk)], idx_v)
            pltpu.async_copy(table_hbm.at[idx_v], rows_v, sem).wait()
            pltpu.sync_copy(rows_v, out_hbm.at[pl.ds(off, chunk)])

    return k(table, idx_flat)


def _deepfm_body(r128_ref, r512_ref, w1_ref, b1_ref, w2_ref, b2_ref,
                 w3_ref, b3_ref, w4_ref, b4_ref, wsel_ref, g_ref,
                 wout_ref, bout_ref, out_ref):
    x = r128_ref[...]
    h = jnp.maximum(jnp.dot(x, w1_ref[...], preferred_element_type=jnp.float32)
                    + b1_ref[...], 0.0)
    h = jnp.maximum(jnp.dot(h, w2_ref[...], preferred_element_type=jnp.float32)
                    + b2_ref[...], 0.0)
    h = jnp.maximum(jnp.dot(h, w3_ref[...], preferred_element_type=jnp.float32)
                    + b3_ref[...], 0.0)
    s8 = jnp.dot(h, w4_ref[...], preferred_element_type=jnp.float32) \
        + b4_ref[...]
    bb = s8.shape[0] // 4
    s_all = jnp.dot(s8, wsel_ref[:, 0:1],
                    preferred_element_type=jnp.float32)
    s_head = jnp.dot(s8, wsel_ref[:, 1:2],
                     preferred_element_type=jnp.float32)
    rowmod = lax.broadcasted_iota(jnp.int32, s_all.shape, 0) % 4
    srow = jnp.where(rowmod == 3, s_head, s_all)
    dnn = jnp.sum(srow.reshape(bb, 4, 1), axis=1)
    xb = r512_ref[...]
    g = g_ref[...]
    sums = jnp.dot(xb, g, preferred_element_type=jnp.float32)
    sumsq = jnp.dot(xb * xb, g, preferred_element_type=jnp.float32)
    fm = 0.5 * (sums * sums - sumsq)
    final = fm + dnn
    z = jnp.dot(final, wout_ref[...], preferred_element_type=jnp.float32) \
        + bout_ref[...]
    out_ref[...] = 1.0 / (1.0 + jnp.exp(-z))


def _tc_deepfm(rows, b, f_pad, W1, b1, W2, b2, W3, b3, W4, b4, Wout, bout):
    e = 16
    eye8 = jnp.eye(8, dtype=jnp.float32)
    w1b = jnp.kron(eye8, W1)
    w2b = jnp.kron(eye8, W2)
    w3b = jnp.kron(eye8, W3)
    w4b = jnp.kron(eye8, W4)
    b1b = jnp.tile(b1, 8).reshape(1, -1)
    b2b = jnp.tile(b2, 8).reshape(1, -1)
    b3b = jnp.tile(b3, 8).reshape(1, -1)
    b4b = jnp.tile(b4, 8).reshape(1, -1)
    wsel = jnp.stack([jnp.ones(8, jnp.float32),
                      (jnp.arange(8) < 2).astype(jnp.float32)], axis=1)
    g = jnp.tile(jnp.eye(e, dtype=jnp.float32), (f_pad, 1)) * \
        (jnp.arange(f_pad * e) // e < 26).astype(jnp.float32)[:, None]

    r128 = rows.reshape(b * f_pad * e // 128, 128)
    r512 = rows.reshape(b, f_pad * e)

    bb = 512
    grid = (b // bb,)
    full = lambda shp: pl.BlockSpec(shp, lambda i: tuple(0 for _ in shp))
    return pl.pallas_call(
        _deepfm_body,
        grid=grid,
        in_specs=[
            pl.BlockSpec((bb * f_pad * e // 128, 128), lambda i: (i, 0)),
            pl.BlockSpec((bb, f_pad * e), lambda i: (i, 0)),
            full(w1b.shape), full(b1b.shape),
            full(w2b.shape), full(b2b.shape),
            full(w3b.shape), full(b3b.shape),
            full(w4b.shape), full(b4b.shape),
            full(wsel.shape), full(g.shape),
            full(Wout.shape), full((1, 1)),
        ],
        out_specs=pl.BlockSpec((bb, 1), lambda i: (i, 0)),
        out_shape=jax.ShapeDtypeStruct((b, 1), jnp.float32),
    )(r128, r512, w1b, b1b, w2b, b2b, w3b, b3b, w4b, b4b, wsel, g,
      Wout, bout.reshape(1, 1))


def kernel(inputs, table, W1, b1, W2, b2, W3, b3, W4, b4, Wout, bout):
    b, f = inputs.shape
    v, e = table.shape
    f_pad = 32
    npad = f_pad - f
    pad = (lax.broadcasted_iota(jnp.int32, (b, npad), 0) * npad
           + lax.broadcasted_iota(jnp.int32, (b, npad), 1)) % v
    idx = jnp.concatenate([inputs.astype(jnp.int32), pad], axis=1)
    idx = idx.reshape(b * f_pad)
    table_lin = _sc_detile(jnp.transpose(table)).reshape(v, e)
    rows = _sc_gather(table_lin, idx)
    return _tc_deepfm(rows, b, f_pad, W1, b1, W2, b2, W3, b3, W4, b4,
                      Wout, bout)


# speedup vs baseline: 1.1672x; 1.1672x over previous
import functools

import jax
import jax.numpy as jnp
from jax import lax
from jax.experimental import pallas as pl
from jax.experimental.pallas import tpu as pltpu
from jax.experimental.pallas import tpu_sc as plsc


def _detile_body(tt_ref, eye_ref, out_ref):
    tt = tt_ref[...]
    e, blk = tt.shape
    y = lax.dot_general(tt, eye_ref[...], (((0,), (0,)), ((), ())),
                        preferred_element_type=jnp.float32)
    y3 = y.reshape(blk // 8, 8, e)
    pieces = [y3[:, c, :] for c in range(8)]
    out_ref[...] = jnp.concatenate(pieces, axis=1)


def _detile(tableT):
    e, v = tableT.shape
    blk = 4096
    grid = (v + blk - 1) // blk
    return pl.pallas_call(
        _detile_body,
        grid=(grid,),
        in_specs=[pl.BlockSpec((e, blk), lambda i: (0, i)),
                  pl.BlockSpec((e, e), lambda i: (0, 0))],
        out_specs=pl.BlockSpec((blk * e // 128, 128), lambda i: (i, 0)),
        out_shape=jax.ShapeDtypeStruct((v * e // 128, 128), jnp.float32),
        compiler_params=pltpu.CompilerParams(fuse_transposed_lhs_in_matmul=True),
    )(tableT, jnp.eye(e, dtype=jnp.float32))


def _sc_gather(table, idx_flat):
    n_total = idx_flat.shape[0]
    v, e = table.shape
    info = plsc.get_sparse_core_info()
    num_cores, num_subcores = info.num_cores, info.num_subcores
    nw = num_cores * num_subcores
    per_w = n_total // nw
    chunk = 4096
    while per_w % chunk:
        chunk //= 2
    n_ch = per_w // chunk

    mesh = plsc.VectorSubcoreMesh(core_axis_name="c", subcore_axis_name="s")

    @functools.partial(
        pl.kernel,
        mesh=mesh,
        out_type=jax.ShapeDtypeStruct((n_total, e), jnp.float32),
        scratch_types=[
            pltpu.VMEM((chunk,), jnp.int32),
            pltpu.VMEM((chunk, e), jnp.float32),
            pltpu.SemaphoreType.DMA,
        ],
        compiler_params=pltpu.CompilerParams(use_tc_tiling_on_sc=False),
    )
    def k(table_hbm, idx_hbm, out_hbm, idx_v, rows_v, sem):
        wid = lax.axis_index("s") * num_cores + lax.axis_index("c")
        base = wid * per_w
        for c in range(n_ch):
            off = base + c * chunk
            pltpu.sync_copy(idx_hbm.at[pl.ds(off, chunk)], idx_v)
            pltpu.async_copy(table_hbm.at[idx_v], rows_v, sem).wait()
            pltpu.sync_copy(rows_v, out_hbm.at[pl.ds(off, chunk)])

    return k(table, idx_flat)


def _deepfm_body(r128_ref, r512_ref, w1_ref, b1_ref, w2_ref, b2_ref,
                 w3_ref, b3_ref, w4_ref, b4_ref, wsel_ref, g_ref,
                 wout_ref, bout_ref, out_ref):
    x = r128_ref[...]
    h = jnp.maximum(jnp.dot(x, w1_ref[...], preferred_element_type=jnp.float32)
                    + b1_ref[...], 0.0)
    h = jnp.maximum(jnp.dot(h, w2_ref[...], preferred_element_type=jnp.float32)
                    + b2_ref[...], 0.0)
    h = jnp.maximum(jnp.dot(h, w3_ref[...], preferred_element_type=jnp.float32)
                    + b3_ref[...], 0.0)
    s8 = jnp.dot(h, w4_ref[...], preferred_element_type=jnp.float32) \
        + b4_ref[...]
    bb = s8.shape[0] // 4
    s_all = jnp.dot(s8, wsel_ref[:, 0:1],
                    preferred_element_type=jnp.float32)
    s_head = jnp.dot(s8, wsel_ref[:, 1:2],
                     preferred_element_type=jnp.float32)
    rowmod = lax.broadcasted_iota(jnp.int32, s_all.shape, 0) % 4
    srow = jnp.where(rowmod == 3, s_head, s_all)
    dnn = jnp.sum(srow.reshape(bb, 4, 1), axis=1)
    xb = r512_ref[...]
    g = g_ref[...]
    sums = jnp.dot(xb, g, preferred_element_type=jnp.float32)
    sumsq = jnp.dot(xb * xb, g, preferred_element_type=jnp.float32)
    fm = 0.5 * (sums * sums - sumsq)
    final = fm + dnn
    z = jnp.dot(final, wout_ref[...], preferred_element_type=jnp.float32) \
        + bout_ref[...]
    out_ref[...] = 1.0 / (1.0 + jnp.exp(-z))


def _tc_deepfm(rows, b, f_pad, W1, b1, W2, b2, W3, b3, W4, b4, Wout, bout):
    e = 16
    eye8 = jnp.eye(8, dtype=jnp.float32)
    w1b = jnp.kron(eye8, W1)
    w2b = jnp.kron(eye8, W2)
    w3b = jnp.kron(eye8, W3)
    w4b = jnp.kron(eye8, W4)
    b1b = jnp.tile(b1, 8).reshape(1, -1)
    b2b = jnp.tile(b2, 8).reshape(1, -1)
    b3b = jnp.tile(b3, 8).reshape(1, -1)
    b4b = jnp.tile(b4, 8).reshape(1, -1)
    wsel = jnp.stack([jnp.ones(8, jnp.float32),
                      (jnp.arange(8) < 2).astype(jnp.float32)], axis=1)
    g = jnp.tile(jnp.eye(e, dtype=jnp.float32), (f_pad, 1)) * \
        (jnp.arange(f_pad * e) // e < 26).astype(jnp.float32)[:, None]

    r128 = rows.reshape(b * f_pad * e // 128, 128)
    r512 = rows.reshape(b, f_pad * e)

    bb = 1024
    grid = (b // bb,)
    full = lambda shp: pl.BlockSpec(shp, lambda i: tuple(0 for _ in shp))
    return pl.pallas_call(
        _deepfm_body,
        grid=grid,
        in_specs=[
            pl.BlockSpec((bb * f_pad * e // 128, 128), lambda i: (i, 0)),
            pl.BlockSpec((bb, f_pad * e), lambda i: (i, 0)),
            full(w1b.shape), full(b1b.shape),
            full(w2b.shape), full(b2b.shape),
            full(w3b.shape), full(b3b.shape),
            full(w4b.shape), full(b4b.shape),
            full(wsel.shape), full(g.shape),
            full(Wout.shape), full((1, 1)),
        ],
        out_specs=pl.BlockSpec((bb, 1), lambda i: (i, 0)),
        out_shape=jax.ShapeDtypeStruct((b, 1), jnp.float32),
    )(r128, r512, w1b, b1b, w2b, b2b, w3b, b3b, w4b, b4b, wsel, g,
      Wout, bout.reshape(1, 1))


def kernel(inputs, table, W1, b1, W2, b2, W3, b3, W4, b4, Wout, bout):
    b, f = inputs.shape
    v, e = table.shape
    f_pad = 32
    npad = f_pad - f
    pad = (lax.broadcasted_iota(jnp.int32, (b, npad), 0) * npad
           + lax.broadcasted_iota(jnp.int32, (b, npad), 1)) % v
    idx = jnp.concatenate([inputs.astype(jnp.int32), pad], axis=1)
    idx = idx.reshape(b * f_pad)
    table_lin = _detile(jnp.transpose(table)).reshape(v, e)
    rows = _sc_gather(table_lin, idx)
    return _tc_deepfm(rows, b, f_pad, W1, b1, W2, b2, W3, b3, W4, b4,
                      Wout, bout)
